# Initial kernel scaffold; baseline (speedup 1.0000x reference)
import jax
import jax.numpy as jnp
from jax.experimental import pallas as pl


def kernel(context, target, neg_samples, W_in, W_out):
    raise NotImplementedError("write your pallas kernel here")


# speedup vs baseline: 4.4842x; 4.4842x over previous
import functools

import jax
import jax.numpy as jnp
from jax import lax
from jax.experimental import pallas as pl
from jax.experimental.pallas import tpu as pltpu
from jax.experimental.pallas import tpu_sc as plsc

VOCAB = 100000
D = 300
B = 16384
CTX = 20
NEG = 20
TN = NEG + 1

NC = 2
NS = 16
NW = NC * NS
BPW = B // NW
G = 2
NG = BPW // G
SCORE_PAD = 32

NCHUNK = 18
TAIL_OFF = 284


def _sc_scores(ctx_idx, tn_idx, w_in, w_out):
    mesh = plsc.VectorSubcoreMesh(
        core_axis_name="c", subcore_axis_name="s", num_cores=NC, num_subcores=NS
    )

    @functools.partial(
        pl.kernel,
        out_type=jax.ShapeDtypeStruct((NW, NG, G * SCORE_PAD), jnp.float32),
        mesh=mesh,
        compiler_params=pltpu.CompilerParams(
            needs_layout_passes=False, use_tc_tiling_on_sc=False
        ),
        scratch_types=[
            pltpu.VMEM((NG, G * CTX), jnp.int32),
            pltpu.VMEM((NG, G * TN), jnp.int32),
            pltpu.VMEM((G * CTX, D), jnp.float32),
            pltpu.VMEM((G * TN, D), jnp.float32),
            pltpu.VMEM((NG, G * SCORE_PAD), jnp.float32),
            pltpu.SemaphoreType.DMA,
            pltpu.SemaphoreType.DMA,
        ],
    )
    def k(ctx_idx_hbm, tn_idx_hbm, w_in_hbm, w_out_hbm, out_hbm,
          ctx_idx_v, tn_idx_v, ctx_rows_v, out_rows_v, scores_v, sem1, sem2):
        wid = lax.axis_index("s") * NC + lax.axis_index("c")
        pltpu.sync_copy(ctx_idx_hbm.at[wid], ctx_idx_v)
        pltpu.sync_copy(tn_idx_hbm.at[wid], tn_idx_v)

        lane = lax.iota(jnp.int32, 16)
        zero = jnp.zeros((16,), jnp.float32)
        inv_ctx = jnp.float32(1.0 / CTX)

        def group(g, carry):
            cp1 = pltpu.async_copy(w_in_hbm.at[ctx_idx_v.at[g]], ctx_rows_v, sem1)
            cp2 = pltpu.async_copy(w_out_hbm.at[tn_idx_v.at[g]], out_rows_v, sem2)
            cp1.wait()
            cp2.wait()
            for e in range(G):
                def ctx_body(r, accs):
                    row = e * CTX + r
                    new = [accs[j] + ctx_rows_v[row, pl.ds(j * 16, 16)]
                           for j in range(NCHUNK)]
                    new.append(accs[NCHUNK] + ctx_rows_v[row, pl.ds(TAIL_OFF, 16)])
                    return tuple(new)

                accs = lax.fori_loop(0, CTX, ctx_body, (zero,) * (NCHUNK + 1))
                ctxc = [a * inv_ctx for a in accs]
                ctxc[NCHUNK] = jnp.where(lane >= 4, ctxc[NCHUNK], 0.0)

                def dot_body(r2, svecs):
                    sv0, sv1 = svecs
                    row = e * TN + r2
                    acc = ctxc[0] * out_rows_v[row, pl.ds(0, 16)]
                    for j in range(1, NCHUNK):
                        acc = acc + ctxc[j] * out_rows_v[row, pl.ds(j * 16, 16)]
                    acc = acc + ctxc[NCHUNK] * out_rows_v[row, pl.ds(TAIL_OFF, 16)]
                    s = jnp.sum(acc)
                    s = jnp.where(r2 == 0, s, -s)
                    sv0 = jnp.where(lane == r2, s, sv0)
                    sv1 = jnp.where(lane == r2 - 16, s, sv1)
                    return sv0, sv1

                sv0, sv1 = lax.fori_loop(0, TN, dot_body, (zero, zero))
                scores_v[g, pl.ds(e * SCORE_PAD, 16)] = sv0
                scores_v[g, pl.ds(e * SCORE_PAD + 16, 16)] = sv1
            return carry

        lax.fori_loop(0, NG, group, 0)
        pltpu.sync_copy(scores_v, out_hbm.at[wid])

    return k(ctx_idx, tn_idx, w_in, w_out)


def _loss_body(x_ref, o_ref):
    x = x_ref[...]
    col = lax.broadcasted_iota(jnp.int32, x.shape, 1) % SCORE_PAD
    valid = col < TN
    ls = jnp.minimum(x, 0.0) - jnp.log1p(jnp.exp(-jnp.abs(x)))
    o_ref[0, 0] = -jnp.sum(jnp.where(valid, ls, 0.0)) * jnp.float32(1.0 / B)


def kernel(context, target, neg_samples, W_in, W_out):
    context = context.astype(jnp.int32)
    target = target.astype(jnp.int32)
    neg_samples = neg_samples.astype(jnp.int32)

    ctx_idx = context.reshape(NW, NG, G * CTX)
    tn = jnp.concatenate([target[:, None], neg_samples], axis=1)
    tn_idx = tn.reshape(NW, NG, G * TN)

    scores = _sc_scores(ctx_idx, tn_idx, W_in, W_out)

    scores2d = scores.reshape(B * SCORE_PAD // 128, 128)
    loss = pl.pallas_call(
        _loss_body,
        out_shape=jax.ShapeDtypeStruct((1, 1), jnp.float32),
        out_specs=pl.BlockSpec(memory_space=pltpu.SMEM),
    )(scores2d)
    return loss[0, 0]
